# Initial kernel scaffold; baseline (speedup 1.0000x reference)
import jax
import jax.numpy as jnp
from jax.experimental import pallas as pl


def kernel(x, i, c, params):
    raise NotImplementedError("write your pallas kernel here")


# speedup vs baseline: 4.8594x; 4.8594x over previous
import math

import jax
import jax.numpy as jnp
import numpy as np
from jax.experimental import pallas as pl
from jax.experimental.pallas import tpu as pltpu

_B = 16
_IN_DIM = 3
_POS_PROJ = 6
_HID = 64
_OUT = 4
_PE = _IN_DIM * 2 * _POS_PROJ
_D_IN = _IN_DIM + _PE
_DIMS = [(_D_IN, _HID), (_HID, _HID), (_HID, _OUT)]
_BLK = 1024

_F32 = jnp.float32
_PREC = jax.lax.Precision.DEFAULT
_PREC_HI = jax.lax.Precision.HIGHEST


def _dt(a, w):
    return jax.lax.dot_general(
        a, w, (((1,), (1,)), ((), ())),
        preferred_element_type=_F32, precision=_PREC)


def _ln(h):
    m = jnp.mean(h, axis=-1, keepdims=True)
    v = jnp.mean((h - m) ** 2, axis=-1, keepdims=True)
    return (h - m) * jax.lax.rsqrt(v + 1e-5)


def _hyper_body(c_ref, *refs):
    ins, outs = refs[:30], refs[30:]
    cc = c_ref[...]
    for l in range(3):
        w0, b0, g0, be0, w1, b1, g1, be1, w2, b2 = ins[l * 10:(l + 1) * 10]
        h = _dt(cc, w0[...]) + b0[...]
        h = jnp.maximum(_ln(h) * g0[...] + be0[...], 0.0)
        h = _dt(h, w1[...]) + b1[...]
        h = jnp.maximum(_ln(h) * g1[...] + be1[...], 0.0)
        outs[l][...] = _dt(h, w2[...]) + b2[...]


def _field_body(bounds_ref, i_ref, x_ref, pm_ref, msk_ref,
                w0a_ref, w0b_ref, b0_ref, w1_ref, b1_ref, w2_ref, b2_ref,
                o_ref):
    t = pl.program_id(0)
    xt = x_ref[...]
    xe = jnp.dot(xt, pm_ref[...], preferred_element_type=_F32,
                 precision=_PREC_HI)
    pe = jnp.where(msk_ref[...] > 0, jnp.sin(xe), jnp.cos(xe))
    iv = i_ref[0]

    s_lo = bounds_ref[t, 0]
    s_hi = bounds_ref[t, 1]

    def body(s, acc):
        h = _dt(xt, w0a_ref[s]) + _dt(pe, w0b_ref[s]) + b0_ref[s]
        h = jnp.maximum(_ln(h), 0.0)
        h = _dt(h, w1_ref[s]) + b1_ref[s]
        h = jnp.maximum(_ln(h), 0.0)
        o = _dt(h, w2_ref[s]) + b2_ref[s]
        return acc + jnp.where(iv == s, o, 0.0)

    o_ref[...] = jax.lax.fori_loop(
        s_lo, s_hi + 1, body, jnp.zeros((_BLK, _OUT), _F32))


def _posemb_consts():
    pm = np.zeros((_IN_DIM, _PE), np.float32)
    msk = np.zeros((1, _PE), np.float32)
    for k in range(_PE):
        d, j = k // (2 * _POS_PROJ), k % (2 * _POS_PROJ)
        pm[d, k] = (2.0 ** (j % _POS_PROJ)) * math.pi
        msk[0, k] = 1.0 if j < _POS_PROJ else 0.0
    return pm, msk


def kernel(x, i, c, params):
    n = x.shape[0]
    b = c.shape[0]
    t = n // _BLK

    plist = []
    for l in range(3):
        for name in ("W0", "b0", "g0", "be0", "W1", "b1", "g1", "be1", "W2", "b2"):
            p = params[f"h{l}_{name}"]
            plist.append(p.reshape(1, -1) if p.ndim == 1 else p)

    nouts = [din * dout + dout for din, dout in _DIMS]
    hp0, hp1, hp2 = pl.pallas_call(
        _hyper_body,
        out_shape=[jax.ShapeDtypeStruct((b, no), _F32) for no in nouts],
    )(c, *plist)

    w0 = hp0[:, :_D_IN * _HID].reshape(b, _HID, _D_IN)
    w0a = w0[:, :, :_IN_DIM]
    w0b = w0[:, :, _IN_DIM:]
    b0 = hp0[:, _D_IN * _HID:].reshape(b, 1, _HID)
    w1 = hp1[:, :_HID * _HID].reshape(b, _HID, _HID)
    b1 = hp1[:, _HID * _HID:].reshape(b, 1, _HID)
    w2 = hp2[:, :_HID * _OUT].reshape(b, _OUT, _HID)
    b2 = hp2[:, _HID * _OUT:].reshape(b, 1, _OUT)

    ii = i.astype(jnp.int32)
    bounds = jnp.stack([ii[::_BLK], ii[_BLK - 1::_BLK]], axis=1)
    i3 = ii.reshape(t, _BLK, 1)
    pm, msk = _posemb_consts()
    pm, msk = jnp.asarray(pm), jnp.asarray(msk)

    full = lambda shape: pl.BlockSpec(shape, lambda tt, bref: (0,) * len(shape))
    out = pl.pallas_call(
        _field_body,
        grid_spec=pltpu.PrefetchScalarGridSpec(
            num_scalar_prefetch=1,
            grid=(t,),
            in_specs=[
                pl.BlockSpec((1, _BLK, 1), lambda tt, bref: (tt, 0, 0)),
                pl.BlockSpec((_BLK, _IN_DIM), lambda tt, bref: (tt, 0)),
                full(pm.shape),
                full(msk.shape),
                full(w0a.shape),
                full(w0b.shape),
                full(b0.shape),
                full(w1.shape),
                full(b1.shape),
                full(w2.shape),
                full(b2.shape),
            ],
            out_specs=pl.BlockSpec((_BLK, _OUT), lambda tt, bref: (tt, 0)),
        ),
        out_shape=jax.ShapeDtypeStruct((n, _OUT), _F32),
    )(bounds, i3, x, pm, msk, w0a, w0b, b0, w1, b1, w2, b2)
    return out


# speedup vs baseline: 15.6175x; 3.2139x over previous
import math

import jax
import jax.numpy as jnp
import numpy as np
from jax.experimental import pallas as pl
from jax.experimental.pallas import tpu as pltpu

_B = 16
_IN_DIM = 3
_POS_PROJ = 6
_HID = 64
_OUT = 4
_PE = _IN_DIM * 2 * _POS_PROJ
_D_IN = _IN_DIM + _PE
_DIMS = [(_D_IN, _HID), (_HID, _HID), (_HID, _OUT)]
_BLK = 1024

_F32 = jnp.float32
_PREC = jax.lax.Precision.DEFAULT


def _dt(a, w):
    return jax.lax.dot_general(
        a, w, (((1,), (1,)), ((), ())),
        preferred_element_type=_F32, precision=_PREC)


def _dm(w, a):
    return jax.lax.dot_general(
        w, a, (((1,), (0,)), ((), ())),
        preferred_element_type=_F32, precision=_PREC)


def _ln(h):
    m = jnp.mean(h, axis=-1, keepdims=True)
    v = jnp.mean((h - m) ** 2, axis=-1, keepdims=True)
    return (h - m) * jax.lax.rsqrt(v + 1e-5)


def _ln0(h):
    m = jnp.mean(h, axis=0, keepdims=True)
    d = h - m
    v = jnp.mean(d * d, axis=0, keepdims=True)
    return d * jax.lax.rsqrt(v + 1e-5)


def _hyper_body(c_ref, *refs):
    ins, outs = refs[:30], refs[30:]
    cc = c_ref[...]
    for l in range(3):
        w0, b0, g0, be0, w1, b1, g1, be1, w2, b2 = ins[l * 10:(l + 1) * 10]
        h = _dt(cc, w0[...]) + b0[...]
        h = jnp.maximum(_ln(h) * g0[...] + be0[...], 0.0)
        h = _dt(h, w1[...]) + b1[...]
        h = jnp.maximum(_ln(h) * g1[...] + be1[...], 0.0)
        outs[l][...] = _dt(h, w2[...]) + b2[...]


def _field_body(bounds_ref, i_ref, x_ref,
                w0_ref, b0_ref, w1_ref, b1_ref, w2_ref, b2_ref,
                o_ref, f_ref):
    t = pl.program_id(0)
    xt = x_ref[...]
    iv = i_ref[0]

    f_ref[0:_IN_DIM, :] = xt
    s = jnp.sin(jnp.float32(math.pi) * xt)
    c = jnp.cos(jnp.float32(math.pi) * xt)
    for j in range(_POS_PROJ):
        lo = _IN_DIM + _IN_DIM * j
        f_ref[lo:lo + _IN_DIM, :] = s
        f_ref[lo + _IN_DIM * _POS_PROJ:lo + _IN_DIM * _POS_PROJ + _IN_DIM, :] = c
        if j < _POS_PROJ - 1:
            s, c = 2.0 * s * c, 1.0 - 2.0 * s * s
    feats = f_ref[...]

    s_lo = bounds_ref[t, 0]
    s_hi = bounds_ref[t, 1]

    def body(seg, acc):
        h = _dm(w0_ref[seg], feats) + b0_ref[seg]
        h = jnp.maximum(_ln0(h), 0.0)
        h = _dm(w1_ref[seg], h) + b1_ref[seg]
        h = jnp.maximum(_ln0(h), 0.0)
        o = _dm(w2_ref[seg], h) + b2_ref[seg]
        return acc + jnp.where(iv == seg, o, 0.0)

    o_ref[...] = jax.lax.fori_loop(
        s_lo, s_hi + 1, body, jnp.zeros((_OUT, _BLK), _F32))


def _feat_perm():
    perm = np.zeros(_D_IN, np.int32)
    perm[0:_IN_DIM] = np.arange(_IN_DIM)
    for j in range(_POS_PROJ):
        for d in range(_IN_DIM):
            perm[_IN_DIM + _IN_DIM * j + d] = _IN_DIM + d * 2 * _POS_PROJ + j
            perm[_IN_DIM + _IN_DIM * (_POS_PROJ + j) + d] = (
                _IN_DIM + d * 2 * _POS_PROJ + _POS_PROJ + j)
    return perm


def kernel(x, i, c, params):
    n = x.shape[0]
    b = c.shape[0]
    t = n // _BLK

    plist = []
    for l in range(3):
        for name in ("W0", "b0", "g0", "be0", "W1", "b1", "g1", "be1", "W2", "b2"):
            p = params[f"h{l}_{name}"]
            plist.append(p.reshape(1, -1) if p.ndim == 1 else p)

    nouts = [din * dout + dout for din, dout in _DIMS]
    hp0, hp1, hp2 = pl.pallas_call(
        _hyper_body,
        out_shape=[jax.ShapeDtypeStruct((b, no), _F32) for no in nouts],
    )(c, *plist)

    w0 = hp0[:, :_D_IN * _HID].reshape(b, _HID, _D_IN)[:, :, _feat_perm()]
    b0 = hp0[:, _D_IN * _HID:].reshape(b, _HID, 1)
    w1 = hp1[:, :_HID * _HID].reshape(b, _HID, _HID)
    b1 = hp1[:, _HID * _HID:].reshape(b, _HID, 1)
    w2 = hp2[:, :_HID * _OUT].reshape(b, _OUT, _HID)
    b2 = hp2[:, _HID * _OUT:].reshape(b, _OUT, 1)

    ii = i.astype(jnp.int32)
    bounds = jnp.stack([ii[::_BLK], ii[_BLK - 1::_BLK]], axis=1)
    i3 = ii.reshape(t, 1, _BLK)
    xt = x.T

    full = lambda shape: pl.BlockSpec(shape, lambda tt, bref: (0,) * len(shape))
    out = pl.pallas_call(
        _field_body,
        grid_spec=pltpu.PrefetchScalarGridSpec(
            num_scalar_prefetch=1,
            grid=(t,),
            in_specs=[
                pl.BlockSpec((1, 1, _BLK), lambda tt, bref: (tt, 0, 0)),
                pl.BlockSpec((_IN_DIM, _BLK), lambda tt, bref: (0, tt)),
                full(w0.shape),
                full(b0.shape),
                full(w1.shape),
                full(b1.shape),
                full(w2.shape),
                full(b2.shape),
            ],
            out_specs=pl.BlockSpec((_OUT, _BLK), lambda tt, bref: (0, tt)),
            scratch_shapes=[pltpu.VMEM((_D_IN, _BLK), _F32)],
        ),
        out_shape=jax.ShapeDtypeStruct((_OUT, n), _F32),
    )(bounds, i3, xt, w0, b0, w1, b1, w2, b2)
    return out.T


# speedup vs baseline: 19.2093x; 1.2300x over previous
import math

import jax
import jax.numpy as jnp
import numpy as np
from jax.experimental import pallas as pl
from jax.experimental.pallas import tpu as pltpu

_B = 16
_IN_DIM = 3
_POS_PROJ = 6
_HID = 64
_OUT = 4
_PE = _IN_DIM * 2 * _POS_PROJ
_D_IN = _IN_DIM + _PE
_DIMS = [(_D_IN, _HID), (_HID, _HID), (_HID, _OUT)]
_BLK = 1024

_F32 = jnp.float32
_PREC = jax.lax.Precision.DEFAULT
_PREC_HI = jax.lax.Precision.HIGHEST


def _dt(a, w, prec=_PREC):
    return jax.lax.dot_general(
        a, w, (((1,), (1,)), ((), ())),
        preferred_element_type=_F32, precision=prec)


def _dm(w, a):
    return jax.lax.dot_general(
        w, a, (((1,), (0,)), ((), ())),
        preferred_element_type=_F32, precision=_PREC)


def _ln(h):
    m = jnp.mean(h, axis=-1, keepdims=True)
    v = jnp.mean((h - m) ** 2, axis=-1, keepdims=True)
    return (h - m) * jax.lax.rsqrt(v + 1e-5)


def _ln0(h):
    m = jnp.mean(h, axis=0, keepdims=True)
    d = h - m
    v = jnp.mean(d * d, axis=0, keepdims=True)
    return d * jax.lax.rsqrt(v + 1e-5)


def _body(bounds_ref, i_ref, x_ref, c_ref, *rest):
    p = rest[:30]
    o_ref = rest[30]
    f_ref = rest[31]
    wb = rest[32:]
    t = pl.program_id(0)

    @pl.when(t == 0)
    def _hyper():
        cc = c_ref[...]
        for l, (din, dout) in enumerate(_DIMS):
            w0, b0, g0, be0, w1, b1, g1, be1, w2, b2 = p[l * 10:(l + 1) * 10]
            wref, bref = wb[2 * l], wb[2 * l + 1]
            h = _dt(cc, w0[...]) + b0[...]
            h = jnp.maximum(_ln(h) * g0[...] + be0[...], 0.0)
            h = _dt(h, w1[...]) + b1[...]
            h = jnp.maximum(_ln(h) * g1[...] + be1[...], 0.0)
            for o in range(dout):
                wsl = w2[o * din:(o + 1) * din, :]
                bsl = b2[0:1, o * din:(o + 1) * din]
                wref[:, o, :] = _dt(h, wsl[...]) + bsl[...]
            bT = _dt(w2[din * dout:din * dout + dout, :], h)
            eye = (jax.lax.broadcasted_iota(jnp.int32, (dout, dout), 0) ==
                   jax.lax.broadcasted_iota(jnp.int32, (dout, dout), 1)
                   ).astype(_F32)
            bcol = _dt(eye, b2[0:1, din * dout:din * dout + dout],
                       prec=_PREC_HI)
            for s in range(_B):
                bref[s] = bT[:, s:s + 1] + bcol

    w0s, b0s, w1s, b1s, w2s, b2s = wb

    xt = x_ref[...]
    f_ref[0:_IN_DIM, :] = xt
    s = jnp.sin(jnp.float32(math.pi) * xt)
    c = jnp.cos(jnp.float32(math.pi) * xt)
    for j in range(_POS_PROJ):
        for d in range(_IN_DIM):
            base = _IN_DIM + d * 2 * _POS_PROJ + j
            f_ref[base:base + 1, :] = s[d:d + 1, :]
            f_ref[base + _POS_PROJ:base + _POS_PROJ + 1, :] = c[d:d + 1, :]
        if j < _POS_PROJ - 1:
            s, c = 2.0 * s * c, 1.0 - 2.0 * s * s
    feats = f_ref[...]

    iv = i_ref[0]
    s_lo = bounds_ref[t, 0]
    s_hi = bounds_ref[t, 1]

    def seg_body(seg, acc):
        h = _dm(w0s[seg], feats) + b0s[seg]
        h = jnp.maximum(_ln0(h), 0.0)
        h = _dm(w1s[seg], h) + b1s[seg]
        h = jnp.maximum(_ln0(h), 0.0)
        o = _dm(w2s[seg], h) + b2s[seg]
        return acc + jnp.where(iv == seg, o, 0.0)

    o_ref[...] = jax.lax.fori_loop(
        s_lo, s_hi + 1, seg_body, jnp.zeros((_OUT, _BLK), _F32))


def kernel(x, i, c, params):
    n = x.shape[0]
    b = c.shape[0]
    t = n // _BLK

    plist = []
    for l in range(3):
        for name in ("W0", "b0", "g0", "be0", "W1", "b1", "g1", "be1", "W2", "b2"):
            pa = params[f"h{l}_{name}"]
            plist.append(pa.reshape(1, -1) if pa.ndim == 1 else pa)

    ii = i.astype(jnp.int32)
    bounds = jnp.stack([ii[::_BLK], ii[_BLK - 1::_BLK]], axis=1)
    i3 = ii.reshape(t, 1, _BLK)
    xt = x.T

    full = lambda shape: pl.BlockSpec(shape, lambda tt, bref: (0,) * len(shape))
    out = pl.pallas_call(
        _body,
        grid_spec=pltpu.PrefetchScalarGridSpec(
            num_scalar_prefetch=1,
            grid=(t,),
            in_specs=[
                pl.BlockSpec((1, 1, _BLK), lambda tt, bref: (tt, 0, 0)),
                pl.BlockSpec((_IN_DIM, _BLK), lambda tt, bref: (0, tt)),
                full(c.shape),
            ] + [full(pa.shape) for pa in plist],
            out_specs=pl.BlockSpec((_OUT, _BLK), lambda tt, bref: (0, tt)),
            scratch_shapes=[
                pltpu.VMEM((_D_IN, _BLK), _F32),
                pltpu.VMEM((b, _HID, _D_IN), _F32),
                pltpu.VMEM((b, _HID, 1), _F32),
                pltpu.VMEM((b, _HID, _HID), _F32),
                pltpu.VMEM((b, _HID, 1), _F32),
                pltpu.VMEM((b, _OUT, _HID), _F32),
                pltpu.VMEM((b, _OUT, 1), _F32),
            ],
        ),
        out_shape=jax.ShapeDtypeStruct((_OUT, n), _F32),
    )(bounds, i3, xt, c, *plist)
    return out.T
